# baseline (speedup 1.0000x reference)
import jax
import jax.numpy as jnp
from jax import lax
from jax.experimental import pallas as pl
from jax.experimental.pallas import tpu as pltpu
from jax.experimental.pallas import tpu_sc as plsc

_B = 1024
_L = 200
_DIM = 64
_PAD = 128
_N = _B * _L
_NC = 2
_NS = 16
_NW = _NC * _NS
_ROWS_PW = _B // _NW
_CHUNK = 200
_NCHUNK = _ROWS_PW
_SUBS = (128, 72)
_UNROLL = 4
_EPS = 1e-12


def _ln_kernel(ids_hbm, word_hbm, pos_hbm, tok_hbm, gam_hbm, bet_hbm,
               out_hbm, idx0_v, idx1_v, rows0_v, rows1_v, out0_v, out1_v,
               pt_v, gam_v, bet_v, tok_v, semg0, semg1, sems0, sems1):
    wid = lax.axis_index("s") * _NC + lax.axis_index("c")
    brow0 = wid * _ROWS_PW

    pltpu.sync_copy(pos_hbm.at[pl.ds(0, _L)], pt_v)
    pltpu.sync_copy(tok_hbm.at[pl.ds(0, 1)], tok_v)
    pltpu.sync_copy(gam_hbm, gam_v)
    pltpu.sync_copy(bet_hbm, bet_v)

    tokq = [tok_v[0, pl.ds(q * 16, 16)] for q in range(4)]

    def add_tok(l, carry):
        for q in range(4):
            sl = pl.ds(q * 16, 16)
            pt_v[l, sl] = pt_v[l, sl] + tokq[q]
        return carry

    lax.fori_loop(0, _L, add_tok, 0)

    gq = [gam_v[pl.ds(q * 16, 16)] for q in range(4)]
    bq = [bet_v[pl.ds(q * 16, 16)] for q in range(4)]
    inv_dim = jnp.float32(1.0 / _DIM)

    def fire_chunk(ci, idx_buf, rows_buf, semg):
        pltpu.sync_copy(ids_hbm.at[pl.ds((brow0 + ci) * _L, _CHUNK)],
                        idx_buf)
        off = 0
        for sub in _SUBS:
            pltpu.async_copy(
                word_hbm.at[idx_buf.at[pl.ds(off, sub)]],
                rows_buf.at[pl.ds(off, sub)],
                semg)
            off += sub

    def drain_chunk(idx_buf, rows_buf, semg):
        off = 0
        for sub in _SUBS:
            pltpu.make_async_copy(
                word_hbm.at[idx_buf.at[pl.ds(off, sub)]],
                rows_buf.at[pl.ds(off, sub)],
                semg).wait()
            off += sub

    def compute_chunk(ci, rows_buf, out_buf):
        def token_body(j, carry):
            for uu in range(_UNROLL):
                t = j * _UNROLL + uu
                c = [rows_buf[t, pl.ds(q * 16, 16)]
                     + pt_v[t, pl.ds(q * 16, 16)] for q in range(4)]
                s = (c[0] + c[1]) + (c[2] + c[3])
                sq = (c[0] * c[0] + c[1] * c[1]) + (c[2] * c[2] + c[3] * c[3])
                tot = jnp.broadcast_to(jnp.sum(s), (16,))
                tot2 = jnp.broadcast_to(jnp.sum(sq), (16,))
                u = tot * inv_dim
                var = tot2 * inv_dim - u * u
                x = var + jnp.float32(_EPS)
                bi = plsc.bitcast(x, jnp.int32)
                bi = jnp.int32(0x5F3759DF) - lax.shift_right_arithmetic(bi, 1)
                r = plsc.bitcast(bi, jnp.float32)
                for _ in range(4):
                    r = r * (jnp.float32(1.5) - jnp.float32(0.5) * x * r * r)
                for q in range(4):
                    out_buf[t, pl.ds(q * 16, 16)] = (c[q] - u) * (r * gq[q]) + bq[q]
            return carry

        lax.fori_loop(0, _CHUNK // _UNROLL, token_body, 0)

    def store_chunk(ci, out_buf, sems):
        pltpu.async_copy(out_buf, out_hbm.at[brow0 + ci], sems)

    def wait_store(out_buf, sems):
        pltpu.make_async_copy(out_buf, out_hbm.at[0], sems).wait()

    fire_chunk(0, idx0_v, rows0_v, semg0)

    def pair_body(i, carry):
        c0 = 2 * i
        c1 = 2 * i + 1
        fire_chunk(c1, idx1_v, rows1_v, semg1)
        drain_chunk(idx0_v, rows0_v, semg0)

        @pl.when(i > 0)
        def _():
            wait_store(out0_v, sems0)

        compute_chunk(c0, rows0_v, out0_v)
        store_chunk(c0, out0_v, sems0)

        @pl.when(i < (_NCHUNK // 2) - 1)
        def _():
            fire_chunk(c1 + 1, idx0_v, rows0_v, semg0)

        drain_chunk(idx1_v, rows1_v, semg1)

        @pl.when(i > 0)
        def _():
            wait_store(out1_v, sems1)

        compute_chunk(c1, rows1_v, out1_v)
        store_chunk(c1, out1_v, sems1)
        return carry

    lax.fori_loop(0, _NCHUNK // 2, pair_body, 0)
    wait_store(out0_v, sems0)
    wait_store(out1_v, sems1)


def _run_fn(ids_flat, word128, pos_emb, tok_emb, gamma, beta):
    run = pl.kernel(
        _ln_kernel,
        out_type=jax.ShapeDtypeStruct((_B, _L, _DIM), jnp.float32),
        mesh=plsc.VectorSubcoreMesh(core_axis_name="c", subcore_axis_name="s"),
        compiler_params=pltpu.CompilerParams(needs_layout_passes=False,
                                             use_tc_tiling_on_sc=True),
        scratch_types=[
            pltpu.VMEM((_CHUNK,), jnp.int32),
            pltpu.VMEM((_CHUNK,), jnp.int32),
            pltpu.VMEM((_CHUNK, _PAD), jnp.float32),
            pltpu.VMEM((_CHUNK, _PAD), jnp.float32),
            pltpu.VMEM((_CHUNK, _DIM), jnp.float32),
            pltpu.VMEM((_CHUNK, _DIM), jnp.float32),
            pltpu.VMEM((_L, _DIM), jnp.float32),
            pltpu.VMEM((_DIM,), jnp.float32),
            pltpu.VMEM((_DIM,), jnp.float32),
            pltpu.VMEM((1, _DIM), jnp.float32),
            pltpu.SemaphoreType.DMA,
            pltpu.SemaphoreType.DMA,
            pltpu.SemaphoreType.DMA,
            pltpu.SemaphoreType.DMA,
        ],
    )
    return run(ids_flat, word128, pos_emb, tok_emb, gamma, beta)


_run_jit = jax.jit(_run_fn)

_VOCAB = 1000000
_PBLK = 2000


def _pad_body(win_ref, wout_ref):
    wout_ref[:, 0:_DIM] = win_ref[...]


def _pad_tc(w):
    return pl.pallas_call(
        _pad_body,
        grid=(_VOCAB // _PBLK,),
        in_specs=[pl.BlockSpec((_PBLK, _DIM), lambda i: (i, 0))],
        out_specs=pl.BlockSpec((_PBLK, _PAD), lambda i: (i, 0)),
        out_shape=jax.ShapeDtypeStruct((_VOCAB, _PAD), jnp.float32),
    )(w)


def kernel(input_ids, word_emb, pos_emb, tok_emb, gamma, beta):
    ids_flat = input_ids.reshape(-1).astype(jnp.int32)
    word128 = _pad_tc(word_emb)
    return _run_jit(ids_flat, word128, pos_emb, tok_emb, gamma, beta)


# speedup vs baseline: 1.4164x; 1.4164x over previous
import jax
import jax.numpy as jnp
from jax import lax
from jax.experimental import pallas as pl
from jax.experimental.pallas import tpu as pltpu
from jax.experimental.pallas import tpu_sc as plsc

_B = 1024
_L = 200
_DIM = 64
_PAD = 128
_N = _B * _L
_NC = 2
_NS = 16
_NW = _NC * _NS
_ROWS_PW = _B // _NW
_CHUNK = 200
_NCHUNK = _ROWS_PW
_SUBS = (128, 72)
_UNROLL = 4
_EPS = 1e-12


def _ln_kernel(ids_hbm, word_hbm, pos_hbm, tok_hbm, gam_hbm, bet_hbm,
               out_hbm, idx0_v, idx1_v, rows0_v, rows1_v, out0_v, out1_v,
               pt_v, gam_v, bet_v, tok_v, semg0, semg1, sems0, sems1):
    wid = lax.axis_index("s") * _NC + lax.axis_index("c")
    brow0 = wid * _ROWS_PW

    pltpu.sync_copy(pos_hbm.at[pl.ds(0, _L)], pt_v)
    pltpu.sync_copy(tok_hbm.at[pl.ds(0, 1)], tok_v)
    pltpu.sync_copy(gam_hbm, gam_v)
    pltpu.sync_copy(bet_hbm, bet_v)

    tokq = [tok_v[0, pl.ds(q * 16, 16)] for q in range(4)]

    def add_tok(l, carry):
        for q in range(4):
            sl = pl.ds(q * 16, 16)
            pt_v[l, sl] = pt_v[l, sl] + tokq[q]
        return carry

    lax.fori_loop(0, _L, add_tok, 0)

    gq = [gam_v[pl.ds(q * 16, 16)] for q in range(4)]
    bq = [bet_v[pl.ds(q * 16, 16)] for q in range(4)]
    inv_dim = jnp.float32(1.0 / _DIM)

    def fire_chunk(ci, idx_buf, rows_buf, semg):
        pltpu.sync_copy(ids_hbm.at[pl.ds((brow0 + ci) * _L, _CHUNK)],
                        idx_buf)
        off = 0
        for sub in _SUBS:
            pltpu.async_copy(
                word_hbm.at[idx_buf.at[pl.ds(off, sub)]],
                rows_buf.at[pl.ds(off, sub)],
                semg)
            off += sub

    def drain_chunk(idx_buf, rows_buf, semg):
        off = 0
        for sub in _SUBS:
            pltpu.make_async_copy(
                word_hbm.at[idx_buf.at[pl.ds(off, sub)]],
                rows_buf.at[pl.ds(off, sub)],
                semg).wait()
            off += sub

    def compute_chunk(ci, rows_buf, out_buf):
        def token_body(j, carry):
            for uu in range(_UNROLL):
                t = j * _UNROLL + uu
                c = [rows_buf[t, pl.ds(q * 16, 16)]
                     + pt_v[t, pl.ds(q * 16, 16)] for q in range(4)]
                s = (c[0] + c[1]) + (c[2] + c[3])
                sq = (c[0] * c[0] + c[1] * c[1]) + (c[2] * c[2] + c[3] * c[3])
                tot = jnp.broadcast_to(jnp.sum(s), (16,))
                tot2 = jnp.broadcast_to(jnp.sum(sq), (16,))
                u = tot * inv_dim
                var = tot2 * inv_dim - u * u
                x = var + jnp.float32(_EPS)
                bi = plsc.bitcast(x, jnp.int32)
                bi = jnp.int32(0x5F3759DF) - lax.shift_right_arithmetic(bi, 1)
                r = plsc.bitcast(bi, jnp.float32)
                for _ in range(4):
                    r = r * (jnp.float32(1.5) - jnp.float32(0.5) * x * r * r)
                for q in range(4):
                    out_buf[t, pl.ds(q * 16, 16)] = (c[q] - u) * (r * gq[q]) + bq[q]
            return carry

        lax.fori_loop(0, _CHUNK // _UNROLL, token_body, 0)

    def store_chunk(ci, out_buf, sems):
        pltpu.async_copy(out_buf, out_hbm.at[brow0 + ci], sems)

    def wait_store(out_buf, sems):
        pltpu.make_async_copy(out_buf, out_hbm.at[0], sems).wait()

    fire_chunk(0, idx0_v, rows0_v, semg0)

    def pair_body(i, carry):
        c0 = 2 * i
        c1 = 2 * i + 1
        fire_chunk(c1, idx1_v, rows1_v, semg1)
        drain_chunk(idx0_v, rows0_v, semg0)

        @pl.when(i > 0)
        def _():
            wait_store(out0_v, sems0)

        compute_chunk(c0, rows0_v, out0_v)
        store_chunk(c0, out0_v, sems0)

        @pl.when(i < (_NCHUNK // 2) - 1)
        def _():
            fire_chunk(c1 + 1, idx0_v, rows0_v, semg0)

        drain_chunk(idx1_v, rows1_v, semg1)

        @pl.when(i > 0)
        def _():
            wait_store(out1_v, sems1)

        compute_chunk(c1, rows1_v, out1_v)
        store_chunk(c1, out1_v, sems1)
        return carry

    lax.fori_loop(0, _NCHUNK // 2, pair_body, 0)
    wait_store(out0_v, sems0)
    wait_store(out1_v, sems1)


def _run_fn(ids_flat, word128, pos_emb, tok_emb, gamma, beta):
    run = pl.kernel(
        _ln_kernel,
        out_type=jax.ShapeDtypeStruct((_B, _L, _DIM), jnp.float32),
        mesh=plsc.VectorSubcoreMesh(core_axis_name="c", subcore_axis_name="s"),
        compiler_params=pltpu.CompilerParams(needs_layout_passes=False,
                                             use_tc_tiling_on_sc=True),
        scratch_types=[
            pltpu.VMEM((_CHUNK,), jnp.int32),
            pltpu.VMEM((_CHUNK,), jnp.int32),
            pltpu.VMEM((_CHUNK, _PAD), jnp.float32),
            pltpu.VMEM((_CHUNK, _PAD), jnp.float32),
            pltpu.VMEM((_CHUNK, _DIM), jnp.float32),
            pltpu.VMEM((_CHUNK, _DIM), jnp.float32),
            pltpu.VMEM((_L, _DIM), jnp.float32),
            pltpu.VMEM((_DIM,), jnp.float32),
            pltpu.VMEM((_DIM,), jnp.float32),
            pltpu.VMEM((1, _DIM), jnp.float32),
            pltpu.SemaphoreType.DMA,
            pltpu.SemaphoreType.DMA,
            pltpu.SemaphoreType.DMA,
            pltpu.SemaphoreType.DMA,
        ],
    )
    return run(ids_flat, word128, pos_emb, tok_emb, gamma, beta)


_run_jit = jax.jit(_run_fn)


def kernel(input_ids, word_emb, pos_emb, tok_emb, gamma, beta):
    ids_flat = input_ids.reshape(-1).astype(jnp.int32)
    word128 = jnp.concatenate(
        [word_emb, jnp.zeros((word_emb.shape[0], _PAD - _DIM), jnp.float32)],
        axis=1)
    return _run_jit(ids_flat, word128, pos_emb, tok_emb, gamma, beta)


# speedup vs baseline: 1.5530x; 1.0965x over previous
import jax
import jax.numpy as jnp
from jax import lax
from jax.experimental import pallas as pl
from jax.experimental.pallas import tpu as pltpu
from jax.experimental.pallas import tpu_sc as plsc

_B = 1024
_L = 200
_DIM = 64
_PAD = 128
_N = _B * _L
_NC = 2
_NS = 16
_NW = _NC * _NS
_ROWS_PW = _B // _NW
_CHUNK = 200
_NCHUNK = _ROWS_PW
_SUBS = (128, 72)
_UNROLL = 4
_EPS = 1e-12


def _ln_kernel(ids_hbm, word_hbm, pos_hbm, tok_hbm, gam_hbm, bet_hbm,
               out_hbm, idx0_v, idx1_v, rows0_v, rows1_v, out0_v, out1_v,
               pt_v, gam_v, bet_v, tok_v, semg0, semg1, sems0, sems1):
    wid = lax.axis_index("s") * _NC + lax.axis_index("c")
    brow0 = wid * _ROWS_PW

    pltpu.sync_copy(pos_hbm.at[pl.ds(0, _L)], pt_v)
    pltpu.sync_copy(tok_hbm.at[pl.ds(0, 1)], tok_v)
    pltpu.sync_copy(gam_hbm, gam_v)
    pltpu.sync_copy(bet_hbm, bet_v)

    tokq = [tok_v[0, pl.ds(q * 16, 16)] for q in range(4)]

    def add_tok(l, carry):
        for q in range(4):
            sl = pl.ds(q * 16, 16)
            pt_v[l, sl] = pt_v[l, sl] + tokq[q]
        return carry

    lax.fori_loop(0, _L, add_tok, 0)

    gq = [gam_v[pl.ds(q * 16, 16)] for q in range(4)]
    bq = [bet_v[pl.ds(q * 16, 16)] for q in range(4)]
    inv_dim = jnp.float32(1.0 / _DIM)

    def fire_chunk(ci, idx_buf, rows_buf, semg):
        pltpu.sync_copy(ids_hbm.at[pl.ds((brow0 + ci) * _L, _CHUNK)],
                        idx_buf)
        off = 0
        for sub in _SUBS:
            pltpu.async_copy(
                word_hbm.at[idx_buf.at[pl.ds(off, sub)]],
                rows_buf.at[pl.ds(off, sub)],
                semg)
            off += sub

    def drain_chunk(idx_buf, rows_buf, semg):
        off = 0
        for sub in _SUBS:
            pltpu.make_async_copy(
                word_hbm.at[idx_buf.at[pl.ds(off, sub)]],
                rows_buf.at[pl.ds(off, sub)],
                semg).wait()
            off += sub

    def compute_chunk(ci, rows_buf, out_buf):
        def token_body(j, carry):
            for uu in range(_UNROLL):
                t = j * _UNROLL + uu
                c = [rows_buf[t, pl.ds(q * 16, 16)]
                     + pt_v[t, pl.ds(q * 16, 16)] for q in range(4)]
                s = (c[0] + c[1]) + (c[2] + c[3])
                sq = (c[0] * c[0] + c[1] * c[1]) + (c[2] * c[2] + c[3] * c[3])
                tot = jnp.broadcast_to(jnp.sum(s), (16,))
                tot2 = jnp.broadcast_to(jnp.sum(sq), (16,))
                u = tot * inv_dim
                var = tot2 * inv_dim - u * u
                x = var + jnp.float32(_EPS)
                bi = plsc.bitcast(x, jnp.int32)
                bi = jnp.int32(0x5F3759DF) - lax.shift_right_arithmetic(bi, 1)
                r = plsc.bitcast(bi, jnp.float32)
                for _ in range(4):
                    r = r * (jnp.float32(1.5) - jnp.float32(0.5) * x * r * r)
                for q in range(4):
                    out_buf[t, pl.ds(q * 16, 16)] = (c[q] - u) * (r * gq[q]) + bq[q]
            return carry

        lax.fori_loop(0, _CHUNK // _UNROLL, token_body, 0)

    def store_chunk(ci, out_buf, sems):
        pltpu.async_copy(out_buf, out_hbm.at[brow0 + ci], sems)

    def wait_store(out_buf, sems):
        pltpu.make_async_copy(out_buf, out_hbm.at[0], sems).wait()

    fire_chunk(0, idx0_v, rows0_v, semg0)

    def pair_body(i, carry):
        c0 = 2 * i
        c1 = 2 * i + 1
        fire_chunk(c1, idx1_v, rows1_v, semg1)
        drain_chunk(idx0_v, rows0_v, semg0)

        @pl.when(i > 0)
        def _():
            wait_store(out0_v, sems0)

        compute_chunk(c0, rows0_v, out0_v)
        store_chunk(c0, out0_v, sems0)

        @pl.when(i < (_NCHUNK // 2) - 1)
        def _():
            fire_chunk(c1 + 1, idx0_v, rows0_v, semg0)

        drain_chunk(idx1_v, rows1_v, semg1)

        @pl.when(i > 0)
        def _():
            wait_store(out1_v, sems1)

        compute_chunk(c1, rows1_v, out1_v)
        store_chunk(c1, out1_v, sems1)
        return carry

    lax.fori_loop(0, _NCHUNK // 2, pair_body, 0)
    wait_store(out0_v, sems0)
    wait_store(out1_v, sems1)


_VOCAB = 1000000
_TBLK = 2048


def _tp_body(wt_ref, out_ref):
    out_ref[:, 0:_DIM] = wt_ref[...].T


def _stage_table(word_emb):
    wt = word_emb.T
    return pl.pallas_call(
        _tp_body,
        grid=(pl.cdiv(_VOCAB, _TBLK),),
        in_specs=[pl.BlockSpec((_DIM, _TBLK), lambda i: (0, i))],
        out_specs=pl.BlockSpec((_TBLK, _PAD), lambda i: (i, 0)),
        out_shape=jax.ShapeDtypeStruct((_VOCAB, _PAD), jnp.float32),
    )(wt)


def _run_fn(ids_flat, word_emb, pos_emb, tok_emb, gamma, beta):
    word128 = _stage_table(word_emb)
    run = pl.kernel(
        _ln_kernel,
        out_type=jax.ShapeDtypeStruct((_B, _L, _DIM), jnp.float32),
        mesh=plsc.VectorSubcoreMesh(core_axis_name="c", subcore_axis_name="s"),
        compiler_params=pltpu.CompilerParams(needs_layout_passes=False,
                                             use_tc_tiling_on_sc=True),
        scratch_types=[
            pltpu.VMEM((_CHUNK,), jnp.int32),
            pltpu.VMEM((_CHUNK,), jnp.int32),
            pltpu.VMEM((_CHUNK, _PAD), jnp.float32),
            pltpu.VMEM((_CHUNK, _PAD), jnp.float32),
            pltpu.VMEM((_CHUNK, _DIM), jnp.float32),
            pltpu.VMEM((_CHUNK, _DIM), jnp.float32),
            pltpu.VMEM((_L, _DIM), jnp.float32),
            pltpu.VMEM((_DIM,), jnp.float32),
            pltpu.VMEM((_DIM,), jnp.float32),
            pltpu.VMEM((1, _DIM), jnp.float32),
            pltpu.SemaphoreType.DMA,
            pltpu.SemaphoreType.DMA,
            pltpu.SemaphoreType.DMA,
            pltpu.SemaphoreType.DMA,
        ],
    )
    return run(ids_flat, word128, pos_emb, tok_emb, gamma, beta)


_run_jit = jax.jit(_run_fn)


def kernel(input_ids, word_emb, pos_emb, tok_emb, gamma, beta):
    ids_flat = input_ids.reshape(-1).astype(jnp.int32)
    return _run_jit(ids_flat, word_emb, pos_emb, tok_emb, gamma, beta)


# speedup vs baseline: 1.8214x; 1.1728x over previous
import jax
import jax.numpy as jnp
from jax import lax
from jax.experimental import pallas as pl
from jax.experimental.pallas import tpu as pltpu
from jax.experimental.pallas import tpu_sc as plsc

_B = 1024
_L = 200
_DIM = 64
_PAD = 128
_N = _B * _L
_NC = 2
_NS = 16
_NW = _NC * _NS
_ROWS_PW = _B // _NW
_CHUNK = 200
_NCHUNK = _ROWS_PW
_SUBS = (128, 72)
_UNROLL = 8
_EPS = 1e-12


def _ln_kernel(ids_hbm, word_hbm, pos_hbm, tok_hbm, gam_hbm, bet_hbm,
               out_hbm, idx0_v, idx1_v, rows0_v, rows1_v, out0_v, out1_v,
               pt_v, gam_v, bet_v, tok_v, semg0, semg1, sems0, sems1):
    wid = lax.axis_index("s") * _NC + lax.axis_index("c")
    brow0 = wid * _ROWS_PW

    pltpu.sync_copy(pos_hbm.at[pl.ds(0, _L)], pt_v)
    pltpu.sync_copy(tok_hbm.at[pl.ds(0, 1)], tok_v)
    pltpu.sync_copy(gam_hbm, gam_v)
    pltpu.sync_copy(bet_hbm, bet_v)

    tokq = [tok_v[0, pl.ds(q * 16, 16)] for q in range(4)]

    def add_tok(l, carry):
        for q in range(4):
            sl = pl.ds(q * 16, 16)
            pt_v[l, sl] = pt_v[l, sl] + tokq[q]
        return carry

    lax.fori_loop(0, _L, add_tok, 0)

    gq = [gam_v[pl.ds(q * 16, 16)] for q in range(4)]
    bq = [bet_v[pl.ds(q * 16, 16)] for q in range(4)]
    inv_dim = jnp.float32(1.0 / _DIM)

    def fire_chunk(ci, idx_buf, rows_buf, semg):
        pltpu.sync_copy(ids_hbm.at[pl.ds((brow0 + ci) * _L, _CHUNK)],
                        idx_buf)
        off = 0
        for sub in _SUBS:
            pltpu.async_copy(
                word_hbm.at[idx_buf.at[pl.ds(off, sub)]],
                rows_buf.at[pl.ds(off, sub)],
                semg)
            off += sub

    def drain_chunk(idx_buf, rows_buf, semg):
        off = 0
        for sub in _SUBS:
            pltpu.make_async_copy(
                word_hbm.at[idx_buf.at[pl.ds(off, sub)]],
                rows_buf.at[pl.ds(off, sub)],
                semg).wait()
            off += sub

    def compute_chunk(ci, rows_buf, out_buf):
        def token_body(j, carry):
            for uu in range(_UNROLL):
                t = j * _UNROLL + uu
                c = [rows_buf[t, pl.ds(q * 16, 16)]
                     + pt_v[t, pl.ds(q * 16, 16)] for q in range(4)]
                s = (c[0] + c[1]) + (c[2] + c[3])
                sq = (c[0] * c[0] + c[1] * c[1]) + (c[2] * c[2] + c[3] * c[3])
                tot = jnp.broadcast_to(jnp.sum(s), (16,))
                tot2 = jnp.broadcast_to(jnp.sum(sq), (16,))
                u = tot * inv_dim
                var = tot2 * inv_dim - u * u
                x = var + jnp.float32(_EPS)
                bi = plsc.bitcast(x, jnp.int32)
                bi = jnp.int32(0x5F3759DF) - lax.shift_right_arithmetic(bi, 1)
                r = plsc.bitcast(bi, jnp.float32)
                for _ in range(4):
                    r = r * (jnp.float32(1.5) - jnp.float32(0.5) * x * r * r)
                for q in range(4):
                    out_buf[t, pl.ds(q * 16, 16)] = (c[q] - u) * (r * gq[q]) + bq[q]
            return carry

        lax.fori_loop(0, _CHUNK // _UNROLL, token_body, 0)

    def store_chunk(ci, out_buf, sems):
        pltpu.async_copy(out_buf, out_hbm.at[brow0 + ci], sems)

    def wait_store(out_buf, sems):
        pltpu.make_async_copy(out_buf, out_hbm.at[0], sems).wait()

    fire_chunk(0, idx0_v, rows0_v, semg0)

    def pair_body(i, carry):
        c0 = 2 * i
        c1 = 2 * i + 1
        fire_chunk(c1, idx1_v, rows1_v, semg1)
        drain_chunk(idx0_v, rows0_v, semg0)

        @pl.when(i > 0)
        def _():
            wait_store(out0_v, sems0)

        compute_chunk(c0, rows0_v, out0_v)
        store_chunk(c0, out0_v, sems0)

        @pl.when(i < (_NCHUNK // 2) - 1)
        def _():
            fire_chunk(c1 + 1, idx0_v, rows0_v, semg0)

        drain_chunk(idx1_v, rows1_v, semg1)

        @pl.when(i > 0)
        def _():
            wait_store(out1_v, sems1)

        compute_chunk(c1, rows1_v, out1_v)
        store_chunk(c1, out1_v, sems1)
        return carry

    lax.fori_loop(0, _NCHUNK // 2, pair_body, 0)
    wait_store(out0_v, sems0)
    wait_store(out1_v, sems1)


_VOCAB = 1000000
_TBLK = 4096


def _tp_body(wt_ref, out_ref):
    out_ref[:, 0:_DIM] = wt_ref[...].T


def _stage_table(word_emb):
    wt = word_emb.T
    return pl.pallas_call(
        _tp_body,
        grid=(pl.cdiv(_VOCAB, _TBLK),),
        in_specs=[pl.BlockSpec((_DIM, _TBLK), lambda i: (0, i))],
        out_specs=pl.BlockSpec((_TBLK, _PAD), lambda i: (i, 0)),
        out_shape=jax.ShapeDtypeStruct((_VOCAB, _PAD), jnp.float32),
    )(wt)


def _run_fn(ids_flat, word_emb, pos_emb, tok_emb, gamma, beta):
    word128 = _stage_table(word_emb)
    run = pl.kernel(
        _ln_kernel,
        out_type=jax.ShapeDtypeStruct((_B, _L, _DIM), jnp.float32),
        mesh=plsc.VectorSubcoreMesh(core_axis_name="c", subcore_axis_name="s"),
        compiler_params=pltpu.CompilerParams(needs_layout_passes=False,
                                             use_tc_tiling_on_sc=True),
        scratch_types=[
            pltpu.VMEM((_CHUNK,), jnp.int32),
            pltpu.VMEM((_CHUNK,), jnp.int32),
            pltpu.VMEM((_CHUNK, _PAD), jnp.float32),
            pltpu.VMEM((_CHUNK, _PAD), jnp.float32),
            pltpu.VMEM((_CHUNK, _DIM), jnp.float32),
            pltpu.VMEM((_CHUNK, _DIM), jnp.float32),
            pltpu.VMEM((_L, _DIM), jnp.float32),
            pltpu.VMEM((_DIM,), jnp.float32),
            pltpu.VMEM((_DIM,), jnp.float32),
            pltpu.VMEM((1, _DIM), jnp.float32),
            pltpu.SemaphoreType.DMA,
            pltpu.SemaphoreType.DMA,
            pltpu.SemaphoreType.DMA,
            pltpu.SemaphoreType.DMA,
        ],
    )
    return run(ids_flat, word128, pos_emb, tok_emb, gamma, beta)


_run_jit = jax.jit(_run_fn)


def kernel(input_ids, word_emb, pos_emb, tok_emb, gamma, beta):
    ids_flat = input_ids.reshape(-1).astype(jnp.int32)
    return _run_jit(ids_flat, word_emb, pos_emb, tok_emb, gamma, beta)


# speedup vs baseline: 2.0928x; 1.1490x over previous
import jax
import jax.numpy as jnp
from jax import lax
from jax.experimental import pallas as pl
from jax.experimental.pallas import tpu as pltpu
from jax.experimental.pallas import tpu_sc as plsc

_B = 1024
_L = 200
_DIM = 64
_PAD = 128
_N = _B * _L
_NC = 2
_NS = 16
_NW = _NC * _NS
_ROWS_PW = _B // _NW
_CHUNK = 200
_NCHUNK = _ROWS_PW
_SUBS = (128, 72)
_UNROLL = 8
_EPS = 1e-12


def _ln_kernel(ids_hbm, word_hbm, pos_hbm, tok_hbm, gam_hbm, bet_hbm,
               out_hbm, idx0_v, idx1_v, rows0_v, rows1_v, out0_v, out1_v,
               pt_v, gam_v, bet_v, tok_v, semg0, semg1, sems0, sems1):
    wid = lax.axis_index("s") * _NC + lax.axis_index("c")
    brow0 = wid * _ROWS_PW

    pltpu.sync_copy(pos_hbm.at[pl.ds(0, _L)], pt_v)
    pltpu.sync_copy(tok_hbm.at[pl.ds(0, 1)], tok_v)
    pltpu.sync_copy(gam_hbm, gam_v)
    pltpu.sync_copy(bet_hbm, bet_v)

    tokq = [tok_v[0, pl.ds(q * 16, 16)] for q in range(4)]

    def add_tok(l, carry):
        for q in range(4):
            sl = pl.ds(q * 16, 16)
            pt_v[l, sl] = pt_v[l, sl] + tokq[q]
        return carry

    lax.fori_loop(0, _L, add_tok, 0)

    gq = [gam_v[pl.ds(q * 16, 16)] for q in range(4)]
    bq = [bet_v[pl.ds(q * 16, 16)] for q in range(4)]
    inv_dim = jnp.float32(1.0 / _DIM)

    def fire_chunk(ci, idx_buf, rows_buf, semg):
        pltpu.sync_copy(ids_hbm.at[pl.ds((brow0 + ci) * _L, _CHUNK)],
                        idx_buf)
        off = 0
        for sub in _SUBS:
            pltpu.async_copy(
                word_hbm.at[idx_buf.at[pl.ds(off, sub)]],
                rows_buf.at[pl.ds(off, sub)],
                semg)
            off += sub

    def drain_chunk(idx_buf, rows_buf, semg):
        off = 0
        for sub in _SUBS:
            pltpu.make_async_copy(
                word_hbm.at[idx_buf.at[pl.ds(off, sub)]],
                rows_buf.at[pl.ds(off, sub)],
                semg).wait()
            off += sub

    def compute_chunk(ci, rows_buf, out_buf):
        def token_body(j, carry):
            for uu in range(_UNROLL):
                t = j * _UNROLL + uu
                c = [rows_buf[t, pl.ds(q * 16, 16)]
                     + pt_v[t, pl.ds(q * 16, 16)] for q in range(4)]
                s = (c[0] + c[1]) + (c[2] + c[3])
                sq = (c[0] * c[0] + c[1] * c[1]) + (c[2] * c[2] + c[3] * c[3])
                tot = jnp.broadcast_to(jnp.sum(s), (16,))
                tot2 = jnp.broadcast_to(jnp.sum(sq), (16,))
                u = tot * inv_dim
                var = tot2 * inv_dim - u * u
                x = var + jnp.float32(_EPS)
                bi = plsc.bitcast(x, jnp.int32)
                bi = jnp.int32(0x5F3759DF) - lax.shift_right_arithmetic(bi, 1)
                r = plsc.bitcast(bi, jnp.float32)
                for _ in range(4):
                    r = r * (jnp.float32(1.5) - jnp.float32(0.5) * x * r * r)
                for q in range(4):
                    out_buf[t, pl.ds(q * 16, 16)] = (c[q] - u) * (r * gq[q]) + bq[q]
            return carry

        lax.fori_loop(0, _CHUNK // _UNROLL, token_body, 0)

    def store_chunk(ci, out_buf, sems):
        pltpu.async_copy(out_buf, out_hbm.at[brow0 + ci], sems)

    def wait_store(out_buf, sems):
        pltpu.make_async_copy(out_buf, out_hbm.at[0], sems).wait()

    fire_chunk(0, idx0_v, rows0_v, semg0)

    def pair_body(i, carry):
        c0 = 2 * i
        c1 = 2 * i + 1
        fire_chunk(c1, idx1_v, rows1_v, semg1)
        drain_chunk(idx0_v, rows0_v, semg0)

        @pl.when(i > 0)
        def _():
            wait_store(out0_v, sems0)

        compute_chunk(c0, rows0_v, out0_v)
        store_chunk(c0, out0_v, sems0)

        @pl.when(i < (_NCHUNK // 2) - 1)
        def _():
            fire_chunk(c1 + 1, idx0_v, rows0_v, semg0)

        drain_chunk(idx1_v, rows1_v, semg1)

        @pl.when(i > 0)
        def _():
            wait_store(out1_v, sems1)

        compute_chunk(c1, rows1_v, out1_v)
        store_chunk(c1, out1_v, sems1)
        return carry

    lax.fori_loop(0, _NCHUNK // 2, pair_body, 0)
    wait_store(out0_v, sems0)
    wait_store(out1_v, sems1)


_VOCAB = 1000000
_TBLK = 8192


def _tp_body(wt_ref, out_ref):
    out_ref[:, 0:_DIM] = wt_ref[...].T


def _stage_table(word_emb):
    wt = word_emb.T
    return pl.pallas_call(
        _tp_body,
        grid=(pl.cdiv(_VOCAB, _TBLK),),
        in_specs=[pl.BlockSpec((_DIM, _TBLK), lambda i: (0, i))],
        out_specs=pl.BlockSpec((_TBLK, _PAD), lambda i: (i, 0)),
        out_shape=jax.ShapeDtypeStruct((_VOCAB, _PAD), jnp.float32),
    )(wt)


def _run_fn(ids_flat, word_emb, pos_emb, tok_emb, gamma, beta):
    word128 = _stage_table(word_emb)
    run = pl.kernel(
        _ln_kernel,
        out_type=jax.ShapeDtypeStruct((_B, _L, _DIM), jnp.float32),
        mesh=plsc.VectorSubcoreMesh(core_axis_name="c", subcore_axis_name="s"),
        compiler_params=pltpu.CompilerParams(needs_layout_passes=False,
                                             use_tc_tiling_on_sc=True),
        scratch_types=[
            pltpu.VMEM((_CHUNK,), jnp.int32),
            pltpu.VMEM((_CHUNK,), jnp.int32),
            pltpu.VMEM((_CHUNK, _PAD), jnp.float32),
            pltpu.VMEM((_CHUNK, _PAD), jnp.float32),
            pltpu.VMEM((_CHUNK, _DIM), jnp.float32),
            pltpu.VMEM((_CHUNK, _DIM), jnp.float32),
            pltpu.VMEM((_L, _DIM), jnp.float32),
            pltpu.VMEM((_DIM,), jnp.float32),
            pltpu.VMEM((_DIM,), jnp.float32),
            pltpu.VMEM((1, _DIM), jnp.float32),
            pltpu.SemaphoreType.DMA,
            pltpu.SemaphoreType.DMA,
            pltpu.SemaphoreType.DMA,
            pltpu.SemaphoreType.DMA,
        ],
    )
    return run(ids_flat, word128, pos_emb, tok_emb, gamma, beta)


_run_jit = jax.jit(_run_fn)


def kernel(input_ids, word_emb, pos_emb, tok_emb, gamma, beta):
    ids_flat = input_ids.reshape(-1).astype(jnp.int32)
    return _run_jit(ids_flat, word_emb, pos_emb, tok_emb, gamma, beta)


# speedup vs baseline: 2.1824x; 1.0428x over previous
import jax
import jax.numpy as jnp
from jax import lax
from jax.experimental import pallas as pl
from jax.experimental.pallas import tpu as pltpu
from jax.experimental.pallas import tpu_sc as plsc

_B = 1024
_L = 200
_DIM = 64
_PAD = 128
_N = _B * _L
_NC = 2
_NS = 16
_NW = _NC * _NS
_ROWS_PW = _B // _NW
_CHUNK = 200
_NCHUNK = _ROWS_PW
_SUBS = (128, 72)
_UNROLL = 8
_EPS = 1e-12


def _ln_kernel(ids_hbm, word_hbm, pos_hbm, tok_hbm, gam_hbm, bet_hbm,
               out_hbm, idx0_v, idx1_v, rows0_v, rows1_v, out0_v, out1_v,
               pt_v, gam_v, bet_v, tok_v, semg0, semg1, sems0, sems1):
    wid = lax.axis_index("s") * _NC + lax.axis_index("c")
    brow0 = wid * _ROWS_PW

    pltpu.sync_copy(pos_hbm.at[pl.ds(0, _L)], pt_v)
    pltpu.sync_copy(tok_hbm.at[pl.ds(0, 1)], tok_v)
    pltpu.sync_copy(gam_hbm, gam_v)
    pltpu.sync_copy(bet_hbm, bet_v)

    tokq = [tok_v[0, pl.ds(q * 16, 16)] for q in range(4)]

    def add_tok(l, carry):
        for q in range(4):
            sl = pl.ds(q * 16, 16)
            pt_v[l, sl] = pt_v[l, sl] + tokq[q]
        return carry

    lax.fori_loop(0, _L, add_tok, 0)

    gq = [gam_v[pl.ds(q * 16, 16)] for q in range(4)]
    bq = [bet_v[pl.ds(q * 16, 16)] for q in range(4)]
    inv_dim = jnp.float32(1.0 / _DIM)

    def fire_chunk(ci, idx_buf, rows_buf, semg):
        pltpu.sync_copy(ids_hbm.at[pl.ds((brow0 + ci) * _L, _CHUNK)],
                        idx_buf)
        off = 0
        for sub in _SUBS:
            pltpu.async_copy(
                word_hbm.at[idx_buf.at[pl.ds(off, sub)]],
                rows_buf.at[pl.ds(off, sub)],
                semg)
            off += sub

    def drain_chunk(idx_buf, rows_buf, semg):
        off = 0
        for sub in _SUBS:
            pltpu.make_async_copy(
                word_hbm.at[idx_buf.at[pl.ds(off, sub)]],
                rows_buf.at[pl.ds(off, sub)],
                semg).wait()
            off += sub

    def compute_chunk(ci, rows_buf, out_buf):
        def token_body(j, carry):
            for uu in range(_UNROLL):
                t = j * _UNROLL + uu
                c = [rows_buf[t, pl.ds(q * 16, 16)]
                     + pt_v[t, pl.ds(q * 16, 16)] for q in range(4)]
                s = (c[0] + c[1]) + (c[2] + c[3])
                sq = (c[0] * c[0] + c[1] * c[1]) + (c[2] * c[2] + c[3] * c[3])
                tot = jnp.broadcast_to(jnp.sum(s), (16,))
                tot2 = jnp.broadcast_to(jnp.sum(sq), (16,))
                u = tot * inv_dim
                var = tot2 * inv_dim - u * u
                x = var + jnp.float32(_EPS)
                bi = plsc.bitcast(x, jnp.int32)
                bi = jnp.int32(0x5F3759DF) - lax.shift_right_arithmetic(bi, 1)
                r = plsc.bitcast(bi, jnp.float32)
                for _ in range(4):
                    r = r * (jnp.float32(1.5) - jnp.float32(0.5) * x * r * r)
                for q in range(4):
                    out_buf[t, pl.ds(q * 16, 16)] = (c[q] - u) * (r * gq[q]) + bq[q]
            return carry

        lax.fori_loop(0, _CHUNK // _UNROLL, token_body, 0)

    def store_chunk(ci, out_buf, sems):
        pltpu.async_copy(out_buf, out_hbm.at[brow0 + ci], sems)

    def wait_store(out_buf, sems):
        pltpu.make_async_copy(out_buf, out_hbm.at[0], sems).wait()

    fire_chunk(0, idx0_v, rows0_v, semg0)

    def pair_body(i, carry):
        c0 = 2 * i
        c1 = 2 * i + 1
        fire_chunk(c1, idx1_v, rows1_v, semg1)
        drain_chunk(idx0_v, rows0_v, semg0)

        @pl.when(i > 0)
        def _():
            wait_store(out0_v, sems0)

        compute_chunk(c0, rows0_v, out0_v)
        store_chunk(c0, out0_v, sems0)

        @pl.when(i < (_NCHUNK // 2) - 1)
        def _():
            fire_chunk(c1 + 1, idx0_v, rows0_v, semg0)

        drain_chunk(idx1_v, rows1_v, semg1)

        @pl.when(i > 0)
        def _():
            wait_store(out1_v, sems1)

        compute_chunk(c1, rows1_v, out1_v)
        store_chunk(c1, out1_v, sems1)
        return carry

    lax.fori_loop(0, _NCHUNK // 2, pair_body, 0)
    wait_store(out0_v, sems0)
    wait_store(out1_v, sems1)


_VOCAB = 1000000
_TBLK = 16384


def _tp_body(wt_ref, out_ref):
    out_ref[:, 0:_DIM] = wt_ref[...].T


def _stage_table(word_emb):
    wt = word_emb.T
    return pl.pallas_call(
        _tp_body,
        grid=(pl.cdiv(_VOCAB, _TBLK),),
        in_specs=[pl.BlockSpec((_DIM, _TBLK), lambda i: (0, i))],
        out_specs=pl.BlockSpec((_TBLK, _PAD), lambda i: (i, 0)),
        out_shape=jax.ShapeDtypeStruct((_VOCAB, _PAD), jnp.float32),
    )(wt)


def _run_fn(ids_flat, word_emb, pos_emb, tok_emb, gamma, beta):
    word128 = _stage_table(word_emb)
    run = pl.kernel(
        _ln_kernel,
        out_type=jax.ShapeDtypeStruct((_B, _L, _DIM), jnp.float32),
        mesh=plsc.VectorSubcoreMesh(core_axis_name="c", subcore_axis_name="s"),
        compiler_params=pltpu.CompilerParams(needs_layout_passes=False,
                                             use_tc_tiling_on_sc=True),
        scratch_types=[
            pltpu.VMEM((_CHUNK,), jnp.int32),
            pltpu.VMEM((_CHUNK,), jnp.int32),
            pltpu.VMEM((_CHUNK, _PAD), jnp.float32),
            pltpu.VMEM((_CHUNK, _PAD), jnp.float32),
            pltpu.VMEM((_CHUNK, _DIM), jnp.float32),
            pltpu.VMEM((_CHUNK, _DIM), jnp.float32),
            pltpu.VMEM((_L, _DIM), jnp.float32),
            pltpu.VMEM((_DIM,), jnp.float32),
            pltpu.VMEM((_DIM,), jnp.float32),
            pltpu.VMEM((1, _DIM), jnp.float32),
            pltpu.SemaphoreType.DMA,
            pltpu.SemaphoreType.DMA,
            pltpu.SemaphoreType.DMA,
            pltpu.SemaphoreType.DMA,
        ],
    )
    return run(ids_flat, word128, pos_emb, tok_emb, gamma, beta)


_run_jit = jax.jit(_run_fn)


def kernel(input_ids, word_emb, pos_emb, tok_emb, gamma, beta):
    ids_flat = input_ids.reshape(-1).astype(jnp.int32)
    return _run_jit(ids_flat, word_emb, pos_emb, tok_emb, gamma, beta)


# speedup vs baseline: 2.2127x; 1.0139x over previous
import jax
import jax.numpy as jnp
from jax import lax
from jax.experimental import pallas as pl
from jax.experimental.pallas import tpu as pltpu
from jax.experimental.pallas import tpu_sc as plsc

_B = 1024
_L = 200
_DIM = 64
_PAD = 128
_N = _B * _L
_NC = 2
_NS = 16
_NW = _NC * _NS
_ROWS_PW = _B // _NW
_CHUNK = 200
_NCHUNK = _ROWS_PW
_SUBS = (128, 72)
_UNROLL = 8
_EPS = 1e-12


def _ln_kernel(ids_hbm, word_hbm, pos_hbm, tok_hbm, gam_hbm, bet_hbm,
               out_hbm, idx0_v, idx1_v, rows0_v, rows1_v, out0_v, out1_v,
               pt_v, gam_v, bet_v, tok_v, semg0, semg1, sems0, sems1):
    wid = lax.axis_index("s") * _NC + lax.axis_index("c")
    brow0 = wid * _ROWS_PW

    pltpu.sync_copy(pos_hbm.at[pl.ds(0, _L)], pt_v)
    pltpu.sync_copy(tok_hbm.at[pl.ds(0, 1)], tok_v)
    pltpu.sync_copy(gam_hbm, gam_v)
    pltpu.sync_copy(bet_hbm, bet_v)

    tokq = [tok_v[0, pl.ds(q * 16, 16)] for q in range(4)]

    def add_tok(l, carry):
        for q in range(4):
            sl = pl.ds(q * 16, 16)
            pt_v[l, sl] = pt_v[l, sl] + tokq[q]
        return carry

    lax.fori_loop(0, _L, add_tok, 0)

    gq = [gam_v[pl.ds(q * 16, 16)] for q in range(4)]
    bq = [bet_v[pl.ds(q * 16, 16)] for q in range(4)]
    inv_dim = jnp.float32(1.0 / _DIM)

    def fire_chunk(ci, idx_buf, rows_buf, semg):
        pltpu.sync_copy(ids_hbm.at[pl.ds((brow0 + ci) * _L, _CHUNK)],
                        idx_buf)
        off = 0
        for sub in _SUBS:
            pltpu.async_copy(
                word_hbm.at[idx_buf.at[pl.ds(off, sub)]],
                rows_buf.at[pl.ds(off, sub)],
                semg)
            off += sub

    def drain_chunk(idx_buf, rows_buf, semg):
        off = 0
        for sub in _SUBS:
            pltpu.make_async_copy(
                word_hbm.at[idx_buf.at[pl.ds(off, sub)]],
                rows_buf.at[pl.ds(off, sub)],
                semg).wait()
            off += sub

    def compute_chunk(ci, rows_buf, out_buf):
        def token_body(j, carry):
            for uu in range(_UNROLL):
                t = j * _UNROLL + uu
                c = [rows_buf[t, pl.ds(q * 16, 16)]
                     + pt_v[t, pl.ds(q * 16, 16)] for q in range(4)]
                s = (c[0] + c[1]) + (c[2] + c[3])
                sq = (c[0] * c[0] + c[1] * c[1]) + (c[2] * c[2] + c[3] * c[3])
                tot = jnp.broadcast_to(jnp.sum(s), (16,))
                tot2 = jnp.broadcast_to(jnp.sum(sq), (16,))
                u = tot * inv_dim
                var = tot2 * inv_dim - u * u
                x = var + jnp.float32(_EPS)
                bi = plsc.bitcast(x, jnp.int32)
                bi = jnp.int32(0x5F3759DF) - lax.shift_right_arithmetic(bi, 1)
                r = plsc.bitcast(bi, jnp.float32)
                for _ in range(4):
                    r = r * (jnp.float32(1.5) - jnp.float32(0.5) * x * r * r)
                for q in range(4):
                    out_buf[t, pl.ds(q * 16, 16)] = (c[q] - u) * (r * gq[q]) + bq[q]
            return carry

        lax.fori_loop(0, _CHUNK // _UNROLL, token_body, 0)

    def store_chunk(ci, out_buf, sems):
        pltpu.async_copy(out_buf, out_hbm.at[brow0 + ci], sems)

    def wait_store(out_buf, sems):
        pltpu.make_async_copy(out_buf, out_hbm.at[0], sems).wait()

    fire_chunk(0, idx0_v, rows0_v, semg0)

    def pair_body(i, carry):
        c0 = 2 * i
        c1 = 2 * i + 1
        fire_chunk(c1, idx1_v, rows1_v, semg1)
        drain_chunk(idx0_v, rows0_v, semg0)

        @pl.when(i > 0)
        def _():
            wait_store(out0_v, sems0)

        compute_chunk(c0, rows0_v, out0_v)
        store_chunk(c0, out0_v, sems0)

        @pl.when(i < (_NCHUNK // 2) - 1)
        def _():
            fire_chunk(c1 + 1, idx0_v, rows0_v, semg0)

        drain_chunk(idx1_v, rows1_v, semg1)

        @pl.when(i > 0)
        def _():
            wait_store(out1_v, sems1)

        compute_chunk(c1, rows1_v, out1_v)
        store_chunk(c1, out1_v, sems1)
        return carry

    lax.fori_loop(0, _NCHUNK // 2, pair_body, 0)
    wait_store(out0_v, sems0)
    wait_store(out1_v, sems1)


_VOCAB = 1000000
_TBLK = 32768


def _tp_body(wt_ref, out_ref):
    out_ref[:, 0:_DIM] = wt_ref[...].T


def _stage_table(word_emb):
    wt = word_emb.T
    return pl.pallas_call(
        _tp_body,
        grid=(pl.cdiv(_VOCAB, _TBLK),),
        in_specs=[pl.BlockSpec((_DIM, _TBLK), lambda i: (0, i))],
        out_specs=pl.BlockSpec((_TBLK, _PAD), lambda i: (i, 0)),
        out_shape=jax.ShapeDtypeStruct((_VOCAB, _PAD), jnp.float32),
    )(wt)


def _run_fn(ids_flat, word_emb, pos_emb, tok_emb, gamma, beta):
    word128 = _stage_table(word_emb)
    run = pl.kernel(
        _ln_kernel,
        out_type=jax.ShapeDtypeStruct((_B, _L, _DIM), jnp.float32),
        mesh=plsc.VectorSubcoreMesh(core_axis_name="c", subcore_axis_name="s"),
        compiler_params=pltpu.CompilerParams(needs_layout_passes=False,
                                             use_tc_tiling_on_sc=True),
        scratch_types=[
            pltpu.VMEM((_CHUNK,), jnp.int32),
            pltpu.VMEM((_CHUNK,), jnp.int32),
            pltpu.VMEM((_CHUNK, _PAD), jnp.float32),
            pltpu.VMEM((_CHUNK, _PAD), jnp.float32),
            pltpu.VMEM((_CHUNK, _DIM), jnp.float32),
            pltpu.VMEM((_CHUNK, _DIM), jnp.float32),
            pltpu.VMEM((_L, _DIM), jnp.float32),
            pltpu.VMEM((_DIM,), jnp.float32),
            pltpu.VMEM((_DIM,), jnp.float32),
            pltpu.VMEM((1, _DIM), jnp.float32),
            pltpu.SemaphoreType.DMA,
            pltpu.SemaphoreType.DMA,
            pltpu.SemaphoreType.DMA,
            pltpu.SemaphoreType.DMA,
        ],
    )
    return run(ids_flat, word128, pos_emb, tok_emb, gamma, beta)


_run_jit = jax.jit(_run_fn)


def kernel(input_ids, word_emb, pos_emb, tok_emb, gamma, beta):
    ids_flat = input_ids.reshape(-1).astype(jnp.int32)
    return _run_jit(ids_flat, word_emb, pos_emb, tok_emb, gamma, beta)


# speedup vs baseline: 2.3642x; 1.0685x over previous
import jax
import jax.numpy as jnp
from jax import lax
from jax.experimental import pallas as pl
from jax.experimental.pallas import tpu as pltpu
from jax.experimental.pallas import tpu_sc as plsc

_B = 1024
_L = 200
_DIM = 64
_PAD = 128
_N = _B * _L
_NC = 2
_NS = 16
_NW = _NC * _NS
_ROWS_PW = _B // _NW
_CHUNK = 200
_NCHUNK = _ROWS_PW
_SUBS = (128, 72)
_UNROLL = 4
_EPS = 1e-12


def _ln_kernel(ids_hbm, word_hbm, pos_hbm, tok_hbm, gam_hbm, bet_hbm,
               out_hbm, idx0_v, idx1_v, rows0_v, rows1_v, out0_v, out1_v,
               pt_v, gam_v, bet_v, tok_v, semg0, semg1, sems0, sems1):
    wid = lax.axis_index("s") * _NC + lax.axis_index("c")
    brow0 = wid * _ROWS_PW

    pltpu.sync_copy(pos_hbm.at[pl.ds(0, _L)], pt_v)
    pltpu.sync_copy(tok_hbm.at[pl.ds(0, 1)], tok_v)
    pltpu.sync_copy(gam_hbm, gam_v)
    pltpu.sync_copy(bet_hbm, bet_v)

    tokq = [tok_v[0, pl.ds(q * 16, 16)] for q in range(4)]

    def add_tok(l, carry):
        for q in range(4):
            sl = pl.ds(q * 16, 16)
            pt_v[l, sl] = pt_v[l, sl] + tokq[q]
        return carry

    lax.fori_loop(0, _L, add_tok, 0)

    gq = [gam_v[pl.ds(q * 16, 16)] for q in range(4)]
    bq = [bet_v[pl.ds(q * 16, 16)] for q in range(4)]
    inv_dim = jnp.float32(1.0 / _DIM)

    def fire_chunk(ci, idx_buf, rows_buf, semg):
        pltpu.sync_copy(ids_hbm.at[pl.ds((brow0 + ci) * _L, _CHUNK)],
                        idx_buf)
        off = 0
        for sub in _SUBS:
            pltpu.async_copy(
                word_hbm.at[idx_buf.at[pl.ds(off, sub)]],
                rows_buf.at[pl.ds(off, sub)],
                semg)
            off += sub

    def drain_chunk(idx_buf, rows_buf, semg):
        off = 0
        for sub in _SUBS:
            pltpu.make_async_copy(
                word_hbm.at[idx_buf.at[pl.ds(off, sub)]],
                rows_buf.at[pl.ds(off, sub)],
                semg).wait()
            off += sub

    def compute_chunk(ci, rows_buf, out_buf):
        def token_body(j, carry):
            for uu in range(_UNROLL):
                t = j * _UNROLL + uu
                c = [rows_buf[t, pl.ds(q * 16, 16)]
                     + pt_v[t, pl.ds(q * 16, 16)] for q in range(4)]
                s = (c[0] + c[1]) + (c[2] + c[3])
                sq = (c[0] * c[0] + c[1] * c[1]) + (c[2] * c[2] + c[3] * c[3])
                tot = jnp.broadcast_to(jnp.sum(s), (16,))
                tot2 = jnp.broadcast_to(jnp.sum(sq), (16,))
                u = tot * inv_dim
                var = tot2 * inv_dim - u * u
                x = var + jnp.float32(_EPS)
                bi = plsc.bitcast(x, jnp.int32)
                bi = jnp.int32(0x5F3759DF) - lax.shift_right_arithmetic(bi, 1)
                r = plsc.bitcast(bi, jnp.float32)
                for _ in range(4):
                    r = r * (jnp.float32(1.5) - jnp.float32(0.5) * x * r * r)
                for q in range(4):
                    out_buf[t, pl.ds(q * 16, 16)] = (c[q] - u) * (r * gq[q]) + bq[q]
            return carry

        lax.fori_loop(0, _CHUNK // _UNROLL, token_body, 0)

    def store_chunk(ci, out_buf, sems):
        pltpu.async_copy(out_buf, out_hbm.at[brow0 + ci], sems)

    def wait_store(out_buf, sems):
        pltpu.make_async_copy(out_buf, out_hbm.at[0], sems).wait()

    fire_chunk(0, idx0_v, rows0_v, semg0)

    def pair_body(i, carry):
        c0 = 2 * i
        c1 = 2 * i + 1
        fire_chunk(c1, idx1_v, rows1_v, semg1)
        drain_chunk(idx0_v, rows0_v, semg0)

        @pl.when(i > 0)
        def _():
            wait_store(out0_v, sems0)

        compute_chunk(c0, rows0_v, out0_v)
        store_chunk(c0, out0_v, sems0)

        @pl.when(i < (_NCHUNK // 2) - 1)
        def _():
            fire_chunk(c1 + 1, idx0_v, rows0_v, semg0)

        drain_chunk(idx1_v, rows1_v, semg1)

        @pl.when(i > 0)
        def _():
            wait_store(out1_v, sems1)

        compute_chunk(c1, rows1_v, out1_v)
        store_chunk(c1, out1_v, sems1)
        return carry

    lax.fori_loop(0, _NCHUNK // 2, pair_body, 0)
    wait_store(out0_v, sems0)
    wait_store(out1_v, sems1)


_VOCAB = 1000000
_TBLK = 32768


def _tp_body(wt_ref, out_ref):
    out_ref[:, 0:_DIM] = wt_ref[...].T


def _stage_table(word_emb):
    wt = word_emb.T
    return pl.pallas_call(
        _tp_body,
        grid=(pl.cdiv(_VOCAB, _TBLK),),
        in_specs=[pl.BlockSpec((_DIM, _TBLK), lambda i: (0, i))],
        out_specs=pl.BlockSpec((_TBLK, _PAD), lambda i: (i, 0)),
        out_shape=jax.ShapeDtypeStruct((_VOCAB, _PAD), jnp.float32),
    )(wt)


def _run_fn(ids_flat, word_emb, pos_emb, tok_emb, gamma, beta):
    word128 = _stage_table(word_emb)
    run = pl.kernel(
        _ln_kernel,
        out_type=jax.ShapeDtypeStruct((_B, _L, _DIM), jnp.float32),
        mesh=plsc.VectorSubcoreMesh(core_axis_name="c", subcore_axis_name="s"),
        compiler_params=pltpu.CompilerParams(needs_layout_passes=False,
                                             use_tc_tiling_on_sc=True),
        scratch_types=[
            pltpu.VMEM((_CHUNK,), jnp.int32),
            pltpu.VMEM((_CHUNK,), jnp.int32),
            pltpu.VMEM((_CHUNK, _PAD), jnp.float32),
            pltpu.VMEM((_CHUNK, _PAD), jnp.float32),
            pltpu.VMEM((_CHUNK, _DIM), jnp.float32),
            pltpu.VMEM((_CHUNK, _DIM), jnp.float32),
            pltpu.VMEM((_L, _DIM), jnp.float32),
            pltpu.VMEM((_DIM,), jnp.float32),
            pltpu.VMEM((_DIM,), jnp.float32),
            pltpu.VMEM((1, _DIM), jnp.float32),
            pltpu.SemaphoreType.DMA,
            pltpu.SemaphoreType.DMA,
            pltpu.SemaphoreType.DMA,
            pltpu.SemaphoreType.DMA,
        ],
    )
    return run(ids_flat, word128, pos_emb, tok_emb, gamma, beta)


_run_jit = jax.jit(_run_fn)


def kernel(input_ids, word_emb, pos_emb, tok_emb, gamma, beta):
    ids_flat = input_ids.reshape(-1).astype(jnp.int32)
    return _run_jit(ids_flat, word_emb, pos_emb, tok_emb, gamma, beta)
